# initial kernel scaffold (unmeasured)
import jax
import jax.numpy as jnp
from jax import lax
from jax.experimental import pallas as pl
from jax.experimental.pallas import tpu as pltpu

N_DEV = 4
SQ = 1024
SKV_SHARD = 1024
HQ_LOCAL = 8
DH = 128
DM = 1024
SCALE = 0.08838834764831843
NEG = -1e9


def _body(x_ref, wq_ref, k_hbm, v_hbm, wo_ref, out_ref,
          kbuf, vbuf, obuf,
          copy_sems, ksend_sems, krecv_sems, vsend_sems, vrecv_sems,
          osend_sems, orecv_sems):
    my = lax.axis_index("i")

    barrier_sem = pltpu.get_barrier_semaphore()
    for d in (1, 2, 3):
        t = lax.rem(my + d, N_DEV)
        pl.semaphore_signal(barrier_sem, inc=1, device_id=(t,),
                            device_id_type=pl.DeviceIdType.MESH)
    pl.semaphore_wait(barrier_sem, 3)

    kown = pltpu.make_async_copy(
        k_hbm.at[0, :, pl.ds(my * HQ_LOCAL, HQ_LOCAL), :],
        kbuf.at[my], copy_sems.at[0])
    vown = pltpu.make_async_copy(
        v_hbm.at[0, :, pl.ds(my * HQ_LOCAL, HQ_LOCAL), :],
        vbuf.at[my], copy_sems.at[1])
    kown.start()
    vown.start()

    sends = []
    for d in (1, 2, 3):
        t = lax.rem(my + d, N_DEV)
        kr = pltpu.make_async_remote_copy(
            src_ref=k_hbm.at[0, :, pl.ds(t * HQ_LOCAL, HQ_LOCAL), :],
            dst_ref=kbuf.at[my],
            send_sem=ksend_sems.at[d - 1],
            recv_sem=krecv_sems.at[my],
            device_id=(t,), device_id_type=pl.DeviceIdType.MESH)
        kr.start()
        vr = pltpu.make_async_remote_copy(
            src_ref=v_hbm.at[0, :, pl.ds(t * HQ_LOCAL, HQ_LOCAL), :],
            dst_ref=vbuf.at[my],
            send_sem=vsend_sems.at[d - 1],
            recv_sem=vrecv_sems.at[my],
            device_id=(t,), device_id_type=pl.DeviceIdType.MESH)
        vr.start()
        sends += [kr, vr]

    q = jnp.dot(x_ref[0], wq_ref[:, :], preferred_element_type=jnp.float32)
    q = (q * SCALE).astype(jnp.bfloat16)

    kown.wait()
    vown.wait()
    for d in (1, 2, 3):
        s = lax.rem(my + (N_DEV - d), N_DEV)
        for buf, rsems, ssems in ((kbuf, krecv_sems, ksend_sems),
                                  (vbuf, vrecv_sems, vsend_sems)):
            pltpu.make_async_remote_copy(
                src_ref=buf.at[s], dst_ref=buf.at[s],
                send_sem=ssems.at[0], recv_sem=rsems.at[s],
                device_id=(s,), device_id_type=pl.DeviceIdType.MESH,
            ).wait_recv()

    acc = [jnp.zeros((SQ, DH), jnp.float32) for _ in range(HQ_LOCAL)]
    lsum = [jnp.zeros((SQ,), jnp.float32) for _ in range(HQ_LOCAL)]
    dims = (((1,), (1,)), ((), ()))

    kc0, vc0 = kbuf[0], vbuf[0]
    qi = lax.broadcasted_iota(jnp.int32, (SQ, SKV_SHARD), 0)
    ki = lax.broadcasted_iota(jnp.int32, (SQ, SKV_SHARD), 1)
    mask0 = (jnp.abs(qi - ki) <= 128) | (ki < 32) | (qi < 32)
    bias0 = jnp.where(mask0, 0.0, NEG).astype(jnp.float32)
    for h in range(HQ_LOCAL):
        qh = q[:, h * DH:(h + 1) * DH]
        sm = lax.dot_general(qh, kc0[:, h, :], dims,
                             preferred_element_type=jnp.float32) + bias0
        p = jnp.exp(sm)
        lsum[h] = lsum[h] + p.sum(axis=1)
        acc[h] = acc[h] + jnp.dot(p.astype(jnp.bfloat16), vc0[:, h, :],
                                  preferred_element_type=jnp.float32)

    for c in (1, 2, 3):
        kc, vc = kbuf[c], vbuf[c]
        for h in range(HQ_LOCAL):
            qh = q[:32, h * DH:(h + 1) * DH]
            sm = lax.dot_general(qh, kc[:, h, :], dims,
                                 preferred_element_type=jnp.float32)
            p = jnp.exp(sm)
            lsum[h] = lsum[h].at[:32].add(p.sum(axis=1))
            acc[h] = acc[h].at[:32, :].add(
                jnp.dot(p.astype(jnp.bfloat16), vc[:, h, :],
                        preferred_element_type=jnp.float32))

    kc1, vc1 = kbuf[1], vbuf[1]
    r = lax.broadcasted_iota(jnp.int32, (DH, DH), 0)
    col = lax.broadcasted_iota(jnp.int32, (DH, DH), 1)
    bias_tri = jnp.where(r >= col, 0.0, NEG).astype(jnp.float32)
    for h in range(HQ_LOCAL):
        qh = q[SQ - 128:, h * DH:(h + 1) * DH]
        sm = lax.dot_general(qh, kc1[:128, h, :], dims,
                             preferred_element_type=jnp.float32) + bias_tri
        p = jnp.exp(sm)
        lsum[h] = lsum[h].at[SQ - 128:].add(p.sum(axis=1))
        acc[h] = acc[h].at[SQ - 128:, :].add(
            jnp.dot(p.astype(jnp.bfloat16), vc1[:128, h, :],
                    preferred_element_type=jnp.float32))

    ctx = jnp.concatenate(
        [(acc[h] / lsum[h][:, None]).astype(jnp.bfloat16)
         for h in range(HQ_LOCAL)], axis=1)
    partial = jnp.dot(ctx, wo_ref[:, :],
                      preferred_element_type=jnp.float32)

    pl.store(obuf, (pl.ds(my, 1), slice(None), slice(None)),
             partial.astype(jnp.bfloat16)[None])
    osends = []
    for d in (1, 2, 3):
        t = lax.rem(my + d, N_DEV)
        o = pltpu.make_async_remote_copy(
            src_ref=obuf.at[my], dst_ref=obuf.at[my],
            send_sem=osend_sems.at[d - 1],
            recv_sem=orecv_sems.at[my],
            device_id=(t,), device_id_type=pl.DeviceIdType.MESH)
        o.start()
        osends.append(o)
    for d in (1, 2, 3):
        s = lax.rem(my + (N_DEV - d), N_DEV)
        pltpu.make_async_remote_copy(
            src_ref=obuf.at[s], dst_ref=obuf.at[s],
            send_sem=osend_sems.at[0], recv_sem=orecv_sems.at[s],
            device_id=(s,), device_id_type=pl.DeviceIdType.MESH,
        ).wait_recv()

    total = (obuf[0].astype(jnp.float32) + obuf[1].astype(jnp.float32)
             + obuf[2].astype(jnp.float32) + obuf[3].astype(jnp.float32))
    out_ref[0, :, :] = total

    for snd in sends + osends:
        snd.wait_send()


def kernel(x, Wq, K_ext, V_ext, Wo):
    xb = x.astype(jnp.bfloat16)
    wqb = Wq.astype(jnp.bfloat16)
    kb = K_ext.astype(jnp.bfloat16)
    vb = V_ext.astype(jnp.bfloat16)
    wob = Wo.astype(jnp.bfloat16)

    return pl.pallas_call(
        _body,
        out_shape=jax.ShapeDtypeStruct((1, SQ, DM), jnp.float32),
        in_specs=[
            pl.BlockSpec(memory_space=pltpu.VMEM),
            pl.BlockSpec(memory_space=pltpu.VMEM),
            pl.BlockSpec(memory_space=pltpu.ANY),
            pl.BlockSpec(memory_space=pltpu.ANY),
            pl.BlockSpec(memory_space=pltpu.VMEM),
        ],
        out_specs=pl.BlockSpec(memory_space=pltpu.VMEM),
        scratch_shapes=[
            pltpu.VMEM((N_DEV, SKV_SHARD, HQ_LOCAL, DH), jnp.bfloat16),
            pltpu.VMEM((N_DEV, SKV_SHARD, HQ_LOCAL, DH), jnp.bfloat16),
            pltpu.VMEM((N_DEV, SQ, DM), jnp.bfloat16),
            pltpu.SemaphoreType.DMA((2,)),
            pltpu.SemaphoreType.DMA((3,)),
            pltpu.SemaphoreType.DMA((N_DEV,)),
            pltpu.SemaphoreType.DMA((3,)),
            pltpu.SemaphoreType.DMA((N_DEV,)),
            pltpu.SemaphoreType.DMA((3,)),
            pltpu.SemaphoreType.DMA((N_DEV,)),
        ],
        compiler_params=pltpu.CompilerParams(collective_id=0),
    )(xb, wqb, kb, vb, wob)


# baseline (device time: 199631 ns/iter reference)
import jax
import jax.numpy as jnp
from jax import lax
from jax.experimental import pallas as pl
from jax.experimental.pallas import tpu as pltpu

N_DEV = 4
SQ = 1024
SKV_SHARD = 1024
HQ_LOCAL = 8
DH = 128
DM = 1024
SCALE = 0.08838834764831843
NEG = -1e9


def _body(x_ref, wq_ref, k_hbm, v_hbm, wo_ref, out_ref,
          kbuf, vbuf, osendbuf, obuf,
          copy_sems, ksend_sems, krecv_sems, vsend_sems, vrecv_sems,
          osend_sems, orecv_sems, agsend_sems, agrecv_sems):
    my = lax.axis_index("i")

    barrier_sem = pltpu.get_barrier_semaphore()
    for d in (1, 2, 3):
        t = lax.rem(my + d, N_DEV)
        pl.semaphore_signal(barrier_sem, inc=1, device_id=(t,),
                            device_id_type=pl.DeviceIdType.MESH)
    pl.semaphore_wait(barrier_sem, 3)

    kown = pltpu.make_async_copy(
        k_hbm.at[0, :, pl.ds(my * HQ_LOCAL, HQ_LOCAL), :],
        kbuf.at[my], copy_sems.at[0])
    vown = pltpu.make_async_copy(
        v_hbm.at[0, :, pl.ds(my * HQ_LOCAL, HQ_LOCAL), :],
        vbuf.at[my], copy_sems.at[1])
    kown.start()
    vown.start()

    sends = []
    for d in (1, 2, 3):
        t = lax.rem(my + d, N_DEV)
        kr = pltpu.make_async_remote_copy(
            src_ref=k_hbm.at[0, :, pl.ds(t * HQ_LOCAL, HQ_LOCAL), :],
            dst_ref=kbuf.at[my],
            send_sem=ksend_sems.at[d - 1],
            recv_sem=krecv_sems.at[my],
            device_id=(t,), device_id_type=pl.DeviceIdType.MESH)
        kr.start()
        vr = pltpu.make_async_remote_copy(
            src_ref=v_hbm.at[0, :, pl.ds(t * HQ_LOCAL, HQ_LOCAL), :],
            dst_ref=vbuf.at[my],
            send_sem=vsend_sems.at[d - 1],
            recv_sem=vrecv_sems.at[my],
            device_id=(t,), device_id_type=pl.DeviceIdType.MESH)
        vr.start()
        sends += [kr, vr]

    q = jnp.dot(x_ref[0], wq_ref[:, :], preferred_element_type=jnp.float32)
    q = (q * SCALE).astype(jnp.bfloat16)

    kown.wait()
    vown.wait()
    for d in (1, 2, 3):
        s = lax.rem(my + (N_DEV - d), N_DEV)
        for buf, rsems, ssems in ((kbuf, krecv_sems, ksend_sems),
                                  (vbuf, vrecv_sems, vsend_sems)):
            pltpu.make_async_remote_copy(
                src_ref=buf.at[s], dst_ref=buf.at[s],
                send_sem=ssems.at[0], recv_sem=rsems.at[s],
                device_id=(s,), device_id_type=pl.DeviceIdType.MESH,
            ).wait_recv()

    dims = (((1,), (1,)), ((), ()))
    RB = 512

    def chunk0_bias(ro):
        qi = lax.broadcasted_iota(jnp.int32, (RB, SKV_SHARD), 0) + ro
        ki = lax.broadcasted_iota(jnp.int32, (RB, SKV_SHARD), 1)
        mask = (jnp.abs(qi - ki) <= 128) | (ki < 32) | (qi < 32)
        return jnp.where(mask, 0.0, NEG).astype(jnp.float32)

    biases = [chunk0_bias(0), chunk0_bias(RB)]

    r = lax.broadcasted_iota(jnp.int32, (DH, DH), 0)
    col = lax.broadcasted_iota(jnp.int32, (DH, DH), 1)
    bias_tri = jnp.where(r >= col, 0.0, NEG).astype(jnp.float32)

    ctx_heads = []
    for h in range(HQ_LOCAL):
        hs = slice(h * DH, (h + 1) * DH)
        k0h = kbuf[0, :, h, :]
        v0h = vbuf[0, :, h, :]
        l0_parts, a0_parts = [], []
        for b in range(SQ // RB):
            sm = lax.dot_general(q[b * RB:(b + 1) * RB, hs], k0h, dims,
                                 preferred_element_type=jnp.float32)
            p = jnp.exp(sm + biases[b])
            l0_parts.append(p.sum(axis=1))
            a0_parts.append(jnp.dot(p.astype(jnp.bfloat16), v0h,
                                    preferred_element_type=jnp.float32))
        l0 = jnp.concatenate(l0_parts)
        a0 = jnp.concatenate(a0_parts, axis=0)

        l32 = jnp.zeros((32,), jnp.float32)
        a32 = jnp.zeros((32, DH), jnp.float32)
        for c in (1, 2, 3):
            sm32 = lax.dot_general(q[:32, hs], kbuf[c, :, h, :], dims,
                                   preferred_element_type=jnp.float32)
            p32 = jnp.exp(sm32)
            l32 = l32 + p32.sum(axis=1)
            a32 = a32 + jnp.dot(p32.astype(jnp.bfloat16), vbuf[c, :, h, :],
                                preferred_element_type=jnp.float32)

        smb = lax.dot_general(q[SQ - 128:, hs], kbuf[1, :128, h, :], dims,
                              preferred_element_type=jnp.float32) + bias_tri
        pb = jnp.exp(smb)
        lb = pb.sum(axis=1)
        ab = jnp.dot(pb.astype(jnp.bfloat16), vbuf[1, :128, h, :],
                     preferred_element_type=jnp.float32)

        lsum = jnp.concatenate(
            [l0[:32] + l32, l0[32:SQ - 128], l0[SQ - 128:] + lb])
        acc = jnp.concatenate(
            [a0[:32] + a32, a0[32:SQ - 128], a0[SQ - 128:] + ab], axis=0)
        ctx_heads.append((acc / lsum[:, None]).astype(jnp.bfloat16))

    ctx = jnp.concatenate(ctx_heads, axis=1)
    partial = jnp.dot(ctx, wo_ref[:, :],
                      preferred_element_type=jnp.float32)

    QR = SQ // N_DEV
    osendbuf[:, :] = partial.astype(jnp.bfloat16)
    rs_sends = []
    for d in (1, 2, 3):
        t = lax.rem(my + d, N_DEV)
        o = pltpu.make_async_remote_copy(
            src_ref=osendbuf.at[pl.ds(t * QR, QR), :],
            dst_ref=obuf.at[d - 1],
            send_sem=osend_sems.at[d - 1],
            recv_sem=orecv_sems.at[d - 1],
            device_id=(t,), device_id_type=pl.DeviceIdType.MESH)
        o.start()
        rs_sends.append(o)
    for d in (1, 2, 3):
        pltpu.make_async_remote_copy(
            src_ref=obuf.at[d - 1], dst_ref=obuf.at[d - 1],
            send_sem=osend_sems.at[0], recv_sem=orecv_sems.at[d - 1],
            device_id=(my,), device_id_type=pl.DeviceIdType.MESH,
        ).wait_recv()

    own_q = osendbuf[pl.ds(my * QR, QR), :].astype(jnp.float32)
    reduced = (own_q + obuf[0].astype(jnp.float32)
               + obuf[1].astype(jnp.float32) + obuf[2].astype(jnp.float32))
    out_ref[0, pl.ds(my * QR, QR), :] = reduced

    ag_sends = []
    for d in (1, 2, 3):
        t = lax.rem(my + d, N_DEV)
        o = pltpu.make_async_remote_copy(
            src_ref=out_ref.at[0, pl.ds(my * QR, QR), :],
            dst_ref=out_ref.at[0, pl.ds(my * QR, QR), :],
            send_sem=agsend_sems.at[d - 1],
            recv_sem=agrecv_sems.at[d - 1],
            device_id=(t,), device_id_type=pl.DeviceIdType.MESH)
        o.start()
        ag_sends.append(o)
    for d in (1, 2, 3):
        s = lax.rem(my + (N_DEV - d), N_DEV)
        pltpu.make_async_remote_copy(
            src_ref=out_ref.at[0, pl.ds(s * QR, QR), :],
            dst_ref=out_ref.at[0, pl.ds(s * QR, QR), :],
            send_sem=agsend_sems.at[0], recv_sem=agrecv_sems.at[d - 1],
            device_id=(s,), device_id_type=pl.DeviceIdType.MESH,
        ).wait_recv()

    for snd in sends + rs_sends + ag_sends:
        snd.wait_send()


def kernel(x, Wq, K_ext, V_ext, Wo):
    xb = x.astype(jnp.bfloat16)
    wqb = Wq.astype(jnp.bfloat16)
    kb = K_ext.astype(jnp.bfloat16)
    vb = V_ext.astype(jnp.bfloat16)
    wob = Wo.astype(jnp.bfloat16)

    return pl.pallas_call(
        _body,
        out_shape=jax.ShapeDtypeStruct((1, SQ, DM), jnp.float32),
        in_specs=[
            pl.BlockSpec(memory_space=pltpu.MemorySpace.VMEM),
            pl.BlockSpec(memory_space=pltpu.MemorySpace.VMEM),
            pl.BlockSpec(memory_space=pltpu.MemorySpace.HBM),
            pl.BlockSpec(memory_space=pltpu.MemorySpace.HBM),
            pl.BlockSpec(memory_space=pltpu.MemorySpace.VMEM),
        ],
        out_specs=pl.BlockSpec(memory_space=pltpu.MemorySpace.VMEM),
        scratch_shapes=[
            pltpu.VMEM((N_DEV, SKV_SHARD, HQ_LOCAL, DH), jnp.bfloat16),
            pltpu.VMEM((N_DEV, SKV_SHARD, HQ_LOCAL, DH), jnp.bfloat16),
            pltpu.VMEM((SQ, DM), jnp.bfloat16),
            pltpu.VMEM((3, SQ // N_DEV, DM), jnp.bfloat16),
            pltpu.SemaphoreType.DMA((2,)),
            pltpu.SemaphoreType.DMA((3,)),
            pltpu.SemaphoreType.DMA((N_DEV,)),
            pltpu.SemaphoreType.DMA((3,)),
            pltpu.SemaphoreType.DMA((N_DEV,)),
            pltpu.SemaphoreType.DMA((3,)),
            pltpu.SemaphoreType.DMA((3,)),
            pltpu.SemaphoreType.DMA((3,)),
            pltpu.SemaphoreType.DMA((3,)),
        ],
        compiler_params=pltpu.CompilerParams(collective_id=0),
    )(xb, wqb, kb, vb, wob)
